# Initial kernel scaffold; baseline (speedup 1.0000x reference)
import jax
import jax.numpy as jnp
from jax.experimental import pallas as pl


def kernel(feats, degree, edge_batch, emb_node_w, emb_degree_w):
    raise NotImplementedError("write your pallas kernel here")


# speedup vs baseline: 1.9647x; 1.9647x over previous
import functools

import jax
import jax.numpy as jnp
from jax import lax
from jax.experimental import pallas as pl
from jax.experimental.pallas import tpu as pltpu
from jax.experimental.pallas import tpu_sc as plsc

N = 100000
D_FEAT = 128
D_NODE = 128
D_ENC = 64
OUT_W = 1 + D_FEAT + D_NODE + D_ENC

NW = 32
B_PAD = 102400
PER_W = B_PAD // NW
CHUNK = 800
N_CHUNKS = PER_W // CHUNK

R = 1000


def _sc_gather(degree_pad, table):
    mesh = plsc.VectorSubcoreMesh(core_axis_name="c", subcore_axis_name="s")

    @functools.partial(
        pl.kernel,
        mesh=mesh,
        out_type=jax.ShapeDtypeStruct((B_PAD, 128), jnp.float32),
        scratch_types=[
            pltpu.VMEM((PER_W,), jnp.int32),
            pltpu.VMEM((CHUNK, 128), jnp.float32),
            pltpu.SemaphoreType.DMA,
        ],
    )
    def k(deg_hbm, table_hbm, out_hbm, idx_v, rows_v, sem):
        wid = lax.axis_index("s") * 2 + lax.axis_index("c")
        base = wid * PER_W
        pltpu.sync_copy(deg_hbm.at[pl.ds(base, PER_W)], idx_v)
        for c in range(N_CHUNKS):
            pltpu.async_copy(
                table_hbm.at[idx_v.at[pl.ds(c * CHUNK, CHUNK)]], rows_v, sem
            ).wait()
            pltpu.sync_copy(rows_v, out_hbm.at[pl.ds(base + c * CHUNK, CHUNK)])

    return k(degree_pad, table)


def _assemble(feats, node_w, deg_emb):
    def body(f_ref, n_ref, d_ref, o_ref):
        z = jnp.zeros((R, 1), jnp.float32)
        o_ref[...] = jnp.concatenate(
            [z, f_ref[...], n_ref[...], d_ref[:, :D_ENC]], axis=1
        )

    return pl.pallas_call(
        body,
        grid=(N // R,),
        in_specs=[
            pl.BlockSpec((R, D_FEAT), lambda i: (i, 0)),
            pl.BlockSpec((R, D_NODE), lambda i: (i, 0)),
            pl.BlockSpec((R, 128), lambda i: (i, 0)),
        ],
        out_specs=pl.BlockSpec((R, OUT_W), lambda i: (i, 0)),
        out_shape=jax.ShapeDtypeStruct((N, OUT_W), jnp.float32),
    )(feats, node_w, deg_emb)


def kernel(feats, degree, edge_batch, emb_node_w, emb_degree_w):
    del edge_batch
    deg_pad = jnp.concatenate(
        [degree.astype(jnp.int32), jnp.zeros((B_PAD - N,), jnp.int32)]
    )
    table_pad = jnp.pad(emb_degree_w, ((0, 0), (0, 128 - D_ENC)))
    deg_emb = _sc_gather(deg_pad, table_pad)
    return _assemble(feats, emb_node_w, deg_emb)


# speedup vs baseline: 1.9851x; 1.0103x over previous
import functools

import jax
import jax.numpy as jnp
from jax import lax
from jax.experimental import pallas as pl
from jax.experimental.pallas import tpu as pltpu
from jax.experimental.pallas import tpu_sc as plsc

N = 100000
D_FEAT = 128
D_NODE = 128
D_ENC = 64
OUT_W = 1 + D_FEAT + D_NODE + D_ENC

NW = 32
B_PAD = 102400
PER_W = B_PAD // NW
CHUNK = 400
N_CHUNKS = PER_W // CHUNK

R = 1000


def _sc_gather(degree_pad, table):
    mesh = plsc.VectorSubcoreMesh(core_axis_name="c", subcore_axis_name="s")

    @functools.partial(
        pl.kernel,
        mesh=mesh,
        out_type=jax.ShapeDtypeStruct((B_PAD, 128), jnp.float32),
        scratch_types=[
            pltpu.VMEM((PER_W,), jnp.int32),
            pltpu.VMEM((CHUNK, 128), jnp.float32),
            pltpu.VMEM((CHUNK, 128), jnp.float32),
            pltpu.SemaphoreType.DMA,
            pltpu.SemaphoreType.DMA,
            pltpu.SemaphoreType.DMA,
            pltpu.SemaphoreType.DMA,
        ],
    )
    def k(deg_hbm, table_hbm, out_hbm, idx_v, rows_a, rows_b, ga, gb, wa, wb):
        wid = lax.axis_index("s") * 2 + lax.axis_index("c")
        base = wid * PER_W
        bufs = (rows_a, rows_b)
        gsems = (ga, gb)
        wsems = (wa, wb)
        pltpu.sync_copy(deg_hbm.at[pl.ds(base, PER_W)], idx_v)

        def gather(c, buf, sem):
            return pltpu.async_copy(
                table_hbm.at[idx_v.at[pl.ds(c * CHUNK, CHUNK)]], buf, sem
            )

        g = [gather(0, bufs[0], gsems[0]), None]
        w = [None, None]
        for c in range(N_CHUNKS):
            b = c % 2
            nb = 1 - b
            if c + 1 < N_CHUNKS:
                if w[nb] is not None:
                    w[nb].wait()
                    w[nb] = None
                g[nb] = gather(c + 1, bufs[nb], gsems[nb])
            g[b].wait()
            w[b] = pltpu.async_copy(
                bufs[b], out_hbm.at[pl.ds(base + c * CHUNK, CHUNK)], wsems[b]
            )
        for b in range(2):
            if w[b] is not None:
                w[b].wait()

    return k(degree_pad, table)


def _assemble(feats, node_w, deg_emb):
    def body(f_ref, n_ref, d_ref, o_ref):
        z = jnp.zeros((R, 1), jnp.float32)
        o_ref[...] = jnp.concatenate(
            [z, f_ref[...], n_ref[...], d_ref[:, :D_ENC]], axis=1
        )

    return pl.pallas_call(
        body,
        grid=(N // R,),
        in_specs=[
            pl.BlockSpec((R, D_FEAT), lambda i: (i, 0)),
            pl.BlockSpec((R, D_NODE), lambda i: (i, 0)),
            pl.BlockSpec((R, 128), lambda i: (i, 0)),
        ],
        out_specs=pl.BlockSpec((R, OUT_W), lambda i: (i, 0)),
        out_shape=jax.ShapeDtypeStruct((N, OUT_W), jnp.float32),
    )(feats, node_w, deg_emb)


def kernel(feats, degree, edge_batch, emb_node_w, emb_degree_w):
    del edge_batch
    deg_pad = jnp.concatenate(
        [degree.astype(jnp.int32), jnp.zeros((B_PAD - N,), jnp.int32)]
    )
    table_pad = jnp.pad(emb_degree_w, ((0, 0), (0, 128 - D_ENC)))
    deg_emb = _sc_gather(deg_pad, table_pad)
    return _assemble(feats, emb_node_w, deg_emb)


# speedup vs baseline: 2.7093x; 1.3649x over previous
import functools

import jax
import jax.numpy as jnp
from jax import lax
from jax.experimental import pallas as pl
from jax.experimental.pallas import tpu as pltpu
from jax.experimental.pallas import tpu_sc as plsc

N = 100000
D_FEAT = 128
D_NODE = 128
D_ENC = 64
OUT_W = 1 + D_FEAT + D_NODE + D_ENC

NW = 32
B_PAD = 102400
PER_W = B_PAD // NW
CHUNK = 400
N_CHUNKS = PER_W // CHUNK

R = 1024


def _sc_gather(degree_pad, table):
    mesh = plsc.VectorSubcoreMesh(core_axis_name="c", subcore_axis_name="s")

    @functools.partial(
        pl.kernel,
        mesh=mesh,
        out_type=jax.ShapeDtypeStruct((B_PAD, 128), jnp.float32),
        scratch_types=[
            pltpu.VMEM((PER_W,), jnp.int32),
            pltpu.VMEM((CHUNK, 128), jnp.float32),
            pltpu.VMEM((CHUNK, 128), jnp.float32),
            pltpu.SemaphoreType.DMA,
            pltpu.SemaphoreType.DMA,
            pltpu.SemaphoreType.DMA,
            pltpu.SemaphoreType.DMA,
        ],
    )
    def k(deg_hbm, table_hbm, out_hbm, idx_v, rows_a, rows_b, ga, gb, wa, wb):
        wid = lax.axis_index("s") * 2 + lax.axis_index("c")
        base = wid * PER_W
        bufs = (rows_a, rows_b)
        gsems = (ga, gb)
        wsems = (wa, wb)
        pltpu.sync_copy(deg_hbm.at[pl.ds(base, PER_W)], idx_v)

        def gather(c, buf, sem):
            return pltpu.async_copy(
                table_hbm.at[idx_v.at[pl.ds(c * CHUNK, CHUNK)]], buf, sem
            )

        g = [gather(0, bufs[0], gsems[0]), None]
        w = [None, None]
        for c in range(N_CHUNKS):
            b = c % 2
            nb = 1 - b
            if c + 1 < N_CHUNKS:
                if w[nb] is not None:
                    w[nb].wait()
                    w[nb] = None
                g[nb] = gather(c + 1, bufs[nb], gsems[nb])
            g[b].wait()
            w[b] = pltpu.async_copy(
                bufs[b], out_hbm.at[pl.ds(base + c * CHUNK, CHUNK)], wsems[b]
            )
        for b in range(2):
            if w[b] is not None:
                w[b].wait()

    return k(degree_pad, table)


def _assemble_t(feats, node_w, deg_emb):
    def body(f_ref, n_ref, d_ref, o_ref):
        z = jnp.zeros((1, R), jnp.float32)
        f_t = jnp.transpose(f_ref[...], (1, 0))
        n_t = jnp.transpose(n_ref[...], (1, 0))
        d_t = jnp.transpose(d_ref[...], (1, 0))
        o_ref[...] = jnp.concatenate([z, f_t, n_t, d_t[:D_ENC]], axis=0)

    return pl.pallas_call(
        body,
        grid=(pl.cdiv(N, R),),
        in_specs=[
            pl.BlockSpec((R, D_FEAT), lambda i: (i, 0)),
            pl.BlockSpec((R, D_NODE), lambda i: (i, 0)),
            pl.BlockSpec((R, 128), lambda i: (i, 0)),
        ],
        out_specs=pl.BlockSpec((OUT_W, R), lambda i: (0, i)),
        out_shape=jax.ShapeDtypeStruct((OUT_W, N), jnp.float32),
    )(feats, node_w, deg_emb)


def kernel(feats, degree, edge_batch, emb_node_w, emb_degree_w):
    del edge_batch
    deg_pad = jnp.concatenate(
        [degree.astype(jnp.int32), jnp.zeros((B_PAD - N,), jnp.int32)]
    )
    table_pad = jnp.pad(emb_degree_w, ((0, 0), (0, 128 - D_ENC)))
    deg_emb = _sc_gather(deg_pad, table_pad)
    return jnp.transpose(_assemble_t(feats, emb_node_w, deg_emb), (1, 0))


# speedup vs baseline: 3.0367x; 1.1208x over previous
import functools

import jax
import jax.numpy as jnp
from jax import lax
from jax.experimental import pallas as pl
from jax.experimental.pallas import tpu as pltpu
from jax.experimental.pallas import tpu_sc as plsc

N = 100000
MAX_DEG_PLUS1 = 513
D_FEAT = 128
D_NODE = 128
D_ENC = 64
OUT_W = 1 + D_FEAT + D_NODE + D_ENC

NW = 32
B_PAD = 102400
PER_W = B_PAD // NW
CHUNK = 400
N_CHUNKS = PER_W // CHUNK

R = 1024


def _sc_gather(degree_pad, table):
    mesh = plsc.VectorSubcoreMesh(core_axis_name="c", subcore_axis_name="s")

    @functools.partial(
        pl.kernel,
        mesh=mesh,
        out_type=jax.ShapeDtypeStruct((B_PAD, 128), jnp.float32),
        scratch_types=[
            pltpu.VMEM((PER_W,), jnp.int32),
            pltpu.VMEM((CHUNK, 128), jnp.float32),
            pltpu.VMEM((CHUNK, 128), jnp.float32),
            pltpu.SemaphoreType.DMA,
            pltpu.SemaphoreType.DMA,
            pltpu.SemaphoreType.DMA,
            pltpu.SemaphoreType.DMA,
        ],
    )
    def k(deg_hbm, table_hbm, out_hbm, idx_v, rows_a, rows_b, ga, gb, wa, wb):
        wid = lax.axis_index("s") * 2 + lax.axis_index("c")
        base = wid * PER_W
        bufs = (rows_a, rows_b)
        gsems = (ga, gb)
        wsems = (wa, wb)
        pltpu.sync_copy(deg_hbm.at[pl.ds(base, PER_W)], idx_v)

        def gather(c, buf, sem):
            return pltpu.async_copy(
                table_hbm.at[idx_v.at[pl.ds(c * CHUNK, CHUNK)]], buf, sem
            )

        g = [gather(0, bufs[0], gsems[0]), None]
        w = [None, None]
        for c in range(N_CHUNKS):
            b = c % 2
            nb = 1 - b
            if c + 1 < N_CHUNKS:
                if w[nb] is not None:
                    w[nb].wait()
                    w[nb] = None
                g[nb] = gather(c + 1, bufs[nb], gsems[nb])
            g[b].wait()
            w[b] = pltpu.async_copy(
                bufs[b], out_hbm.at[pl.ds(base + c * CHUNK, CHUNK)], wsems[b]
            )
        for b in range(2):
            if w[b] is not None:
                w[b].wait()

    return k(degree_pad, table)


def _assemble_t(feats, node_w, deg_emb):
    def body(f_ref, n_ref, d_ref, o_ref):
        z = jnp.zeros((1, R), jnp.float32)
        f_t = jnp.transpose(f_ref[...], (1, 0))
        n_t = jnp.transpose(n_ref[...], (1, 0))
        d_t = jnp.transpose(d_ref[...], (1, 0))
        o_ref[...] = jnp.concatenate([z, f_t, n_t, d_t[:D_ENC]], axis=0)

    return pl.pallas_call(
        body,
        grid=(pl.cdiv(N, R),),
        in_specs=[
            pl.BlockSpec((R, D_FEAT), lambda i: (i, 0)),
            pl.BlockSpec((R, D_NODE), lambda i: (i, 0)),
            pl.BlockSpec((R, 128), lambda i: (i, 0)),
        ],
        out_specs=pl.BlockSpec((OUT_W, R), lambda i: (0, i)),
        out_shape=jax.ShapeDtypeStruct((OUT_W, N), jnp.float32),
    )(feats, node_w, deg_emb)


def kernel(feats, degree, edge_batch, emb_node_w, emb_degree_w):
    del edge_batch
    deg_pad = jnp.concatenate(
        [degree.astype(jnp.int32), jnp.zeros((B_PAD - N,), jnp.int32)]
    )
    table_pad = jnp.pad(emb_degree_w, ((0, 0), (0, 128 - D_ENC)))
    table_rep = jnp.tile(table_pad, (NW, 1))
    offs = (jnp.arange(B_PAD, dtype=jnp.int32) // PER_W) * MAX_DEG_PLUS1
    deg_emb = _sc_gather(deg_pad + offs, table_rep)
    return jnp.transpose(_assemble_t(feats, emb_node_w, deg_emb), (1, 0))


# speedup vs baseline: 3.7738x; 1.2427x over previous
import functools

import jax
import jax.numpy as jnp
from jax import lax
from jax.experimental import pallas as pl
from jax.experimental.pallas import tpu as pltpu
from jax.experimental.pallas import tpu_sc as plsc

N = 100000
MAX_DEG_PLUS1 = 513
D_FEAT = 128
D_NODE = 128
D_ENC = 64
OUT_W = 1 + D_FEAT + D_NODE + D_ENC

NW = 32
N_SUB = 16
CHUNK = 400
A_ROWS = 5200
B_ROWS = 1200
PAIR_ROWS = A_ROWS + B_ROWS
B_PAD = N_SUB * PAIR_ROWS

R = 1024


def _sc_gather(degree_pad, table):
    mesh = plsc.VectorSubcoreMesh(core_axis_name="c", subcore_axis_name="s")

    @functools.partial(
        pl.kernel,
        mesh=mesh,
        out_type=jax.ShapeDtypeStruct((B_PAD, 128), jnp.float32),
        scratch_types=[
            pltpu.VMEM((A_ROWS,), jnp.int32),
            pltpu.VMEM((CHUNK, 128), jnp.float32),
            pltpu.VMEM((CHUNK, 128), jnp.float32),
            pltpu.SemaphoreType.DMA,
            pltpu.SemaphoreType.DMA,
            pltpu.SemaphoreType.DMA,
            pltpu.SemaphoreType.DMA,
        ],
    )
    def k(deg_hbm, table_hbm, out_hbm, idx_v, rows_a, rows_b, ga, gb, wa, wb):
        c = lax.axis_index("c")
        s = lax.axis_index("s")
        base = s * PAIR_ROWS + c * A_ROWS
        bufs = (rows_a, rows_b)
        gsems = (ga, gb)
        wsems = (wa, wb)

        def run(nrows):
            nchunks = nrows // CHUNK
            pltpu.sync_copy(
                deg_hbm.at[pl.ds(base, nrows)], idx_v.at[pl.ds(0, nrows)]
            )

            def gather(ci, buf, sem):
                return pltpu.async_copy(
                    table_hbm.at[idx_v.at[pl.ds(ci * CHUNK, CHUNK)]], buf, sem
                )

            g = [gather(0, bufs[0], gsems[0]), None]
            w = [None, None]
            for ci in range(nchunks):
                b = ci % 2
                nb = 1 - b
                if ci + 1 < nchunks:
                    if w[nb] is not None:
                        w[nb].wait()
                        w[nb] = None
                    g[nb] = gather(ci + 1, bufs[nb], gsems[nb])
                g[b].wait()
                w[b] = pltpu.async_copy(
                    bufs[b], out_hbm.at[pl.ds(base + ci * CHUNK, CHUNK)], wsems[b]
                )
            for b in range(2):
                if w[b] is not None:
                    w[b].wait()

        @pl.when(c == 0)
        def _():
            run(A_ROWS)

        @pl.when(c == 1)
        def _():
            run(B_ROWS)

    return k(degree_pad, table)


def _assemble_t(feats, node_w, deg_emb):
    def body(f_ref, n_ref, d_ref, o_ref):
        z = jnp.zeros((1, R), jnp.float32)
        f_t = jnp.transpose(f_ref[...], (1, 0))
        n_t = jnp.transpose(n_ref[...], (1, 0))
        d_t = jnp.transpose(d_ref[...], (1, 0))
        o_ref[...] = jnp.concatenate([z, f_t, n_t, d_t[:D_ENC]], axis=0)

    return pl.pallas_call(
        body,
        grid=(pl.cdiv(N, R),),
        in_specs=[
            pl.BlockSpec((R, D_FEAT), lambda i: (i, 0)),
            pl.BlockSpec((R, D_NODE), lambda i: (i, 0)),
            pl.BlockSpec((R, 128), lambda i: (i, 0)),
        ],
        out_specs=pl.BlockSpec((OUT_W, R), lambda i: (0, i)),
        out_shape=jax.ShapeDtypeStruct((OUT_W, N), jnp.float32),
    )(feats, node_w, deg_emb)


def kernel(feats, degree, edge_batch, emb_node_w, emb_degree_w):
    del edge_batch
    deg_pad = jnp.concatenate(
        [degree.astype(jnp.int32), jnp.zeros((B_PAD - N,), jnp.int32)]
    )
    table_pad = jnp.pad(emb_degree_w, ((0, 0), (0, 128 - D_ENC)))
    table_rep = jnp.tile(table_pad, (NW, 1))
    r = jnp.arange(B_PAD, dtype=jnp.int32)
    s = r // PAIR_ROWS
    cc = ((r % PAIR_ROWS) >= A_ROWS).astype(jnp.int32)
    offs = (s * 2 + cc) * MAX_DEG_PLUS1
    deg_emb = _sc_gather(deg_pad + offs, table_rep)
    return jnp.transpose(_assemble_t(feats, emb_node_w, deg_emb), (1, 0))
